# baseline (speedup 1.0000x reference)
import functools

import jax
import jax.numpy as jnp
from jax import lax
from jax.experimental import pallas as pl
from jax.experimental.pallas import tpu as pltpu
from jax.experimental.pallas import tpu_sc as plsc

_DELTA = 10
_ACTIONS = 512
_EMB = 32
_PLANES = 2 * _DELTA + 1

_NC = 2
_NS = 16
_NW = _NC * _NS
_AG = _ACTIONS // _NW
_NB = 64


_PAD = _EMB + 1


def _spline_body(x_hbm, tbl_hbm, out_hbm, tbl_v, x_v, out_v):
    wid = lax.axis_index("s") * _NC + lax.axis_index("c")
    a0 = wid * _AG
    n_total = x_hbm.shape[0]

    pltpu.sync_copy(tbl_hbm.at[:, pl.ds(a0, _AG), :], tbl_v.at[:, :, pl.ds(0, _EMB)])

    iota = lax.iota(jnp.int32, _AG)
    inv_d = jnp.float32(1.0 / _DELTA)

    @pl.loop(0, n_total // _NB)
    def _block(b):
        n0 = b * _NB
        pltpu.sync_copy(x_hbm.at[pl.ds(n0, _NB), pl.ds(a0, _AG)], x_v)

        iota_pad = iota * _PAD
        zv = jnp.zeros((_AG,), jnp.int32)

        @plsc.parallel_loop(0, _NB, unroll=2)
        def _row(i):
            iv = jnp.full((_AG,), i, jnp.int32)
            xi = plsc.load_gather(x_v, [iv, iota])
            t = xi * jnp.float32(_DELTA)
            xl_i = t.astype(jnp.int32)
            xh_i = (t + 1.0).astype(jnp.float32).astype(jnp.int32)
            xl_s = xl_i.astype(jnp.float32) / jnp.float32(_DELTA)
            xh_s = xh_i.astype(jnp.float32) / jnp.float32(_DELTA)
            w_h = (xi - xl_s) / inv_d
            w_l = (xh_s - xi) / inv_d
            plane_l = xl_i + _DELTA
            plane_h = jnp.minimum(xh_i + _DELTA, _PLANES - 1)

            base_l = plane_l * (_AG * _PAD) + iota_pad
            base_h = plane_h * (_AG * _PAD) + iota_pad
            base_o = i * (_AG * _PAD) + iota_pad

            for e in range(_EMB):
                bl = plsc.load_gather(tbl_v, [zv, zv, base_l + e])
                bh = plsc.load_gather(tbl_v, [zv, zv, base_h + e])
                val = bh * w_h + bl * w_l
                plsc.store_scatter(out_v, [zv, zv, base_o + e], val)

        pltpu.sync_copy(
            out_v.at[:, :, pl.ds(0, _EMB)],
            out_hbm.at[pl.ds(n0, _NB), pl.ds(a0, _AG), :],
        )


@jax.jit
def kernel(x, table):
    n = x.shape[0]
    tbl3 = table.reshape(_PLANES, _ACTIONS, _EMB)
    mesh = plsc.VectorSubcoreMesh(core_axis_name="c", subcore_axis_name="s")
    run = pl.kernel(
        _spline_body,
        out_type=jax.ShapeDtypeStruct((n, _ACTIONS, _EMB), jnp.float32),
        mesh=mesh,
        scratch_types=[
            pltpu.VMEM((_PLANES, _AG, _PAD), jnp.float32),
            pltpu.VMEM((_NB, _AG), jnp.float32),
            pltpu.VMEM((_NB, _AG, _PAD), jnp.float32),
        ],
        compiler_params=pltpu.CompilerParams(
            use_tc_tiling_on_sc=False, needs_layout_passes=False
        ),
    )
    return run(x, tbl3)


# speedup vs baseline: 1.1775x; 1.1775x over previous
import functools

import jax
import jax.numpy as jnp
from jax import lax
from jax.experimental import pallas as pl
from jax.experimental.pallas import tpu as pltpu
from jax.experimental.pallas import tpu_sc as plsc

_DELTA = 10
_ACTIONS = 512
_EMB = 32
_PLANES = 2 * _DELTA + 1

_NC = 2
_NS = 16
_NW = _NC * _NS
_AG = _ACTIONS // _NW
_NB = 64


_PAD = _EMB + 1


def _spline_body(x_hbm, tbl_hbm, out_hbm, tbl_v, x_v, out_v):
    wid = lax.axis_index("s") * _NC + lax.axis_index("c")
    a0 = wid * _AG
    n_total = x_hbm.shape[0]

    pltpu.sync_copy(tbl_hbm.at[:, pl.ds(a0, _AG), :], tbl_v.at[:, :, pl.ds(0, _EMB)])

    iota = lax.iota(jnp.int32, _AG)
    inv_d = jnp.float32(1.0 / _DELTA)

    @pl.loop(0, n_total // _NB)
    def _block(b):
        n0 = b * _NB
        pltpu.sync_copy(x_hbm.at[pl.ds(n0, _NB), pl.ds(a0, _AG)], x_v)

        @plsc.parallel_loop(0, _NB, unroll=2)
        def _row(i):
            iv = jnp.full((_AG,), i, jnp.int32)
            xi = plsc.load_gather(x_v, [iv, iota])
            t = xi * jnp.float32(_DELTA)
            xl_i = t.astype(jnp.int32)
            xh_i = (t + 1.0).astype(jnp.float32).astype(jnp.int32)
            xl_s = xl_i.astype(jnp.float32) / jnp.float32(_DELTA)
            xh_s = xh_i.astype(jnp.float32) / jnp.float32(_DELTA)
            w_h = (xi - xl_s) / inv_d
            w_l = (xh_s - xi) / inv_d
            plane_l = xl_i + _DELTA
            plane_h = jnp.minimum(xh_i + _DELTA, _PLANES - 1)

            @plsc.parallel_loop(0, _EMB, unroll=8)
            def _emb(e):
                ev = jnp.full((_AG,), e, jnp.int32)
                bl = plsc.load_gather(tbl_v, [plane_l, iota, ev])
                bh = plsc.load_gather(tbl_v, [plane_h, iota, ev])
                val = bh * w_h + bl * w_l
                plsc.store_scatter(out_v, [iv, iota, ev], val)

        pltpu.sync_copy(
            out_v.at[:, :, pl.ds(0, _EMB)],
            out_hbm.at[pl.ds(n0, _NB), pl.ds(a0, _AG), :],
        )


@jax.jit
def kernel(x, table):
    n = x.shape[0]
    tbl3 = table.reshape(_PLANES, _ACTIONS, _EMB)
    mesh = plsc.VectorSubcoreMesh(core_axis_name="c", subcore_axis_name="s")
    run = pl.kernel(
        _spline_body,
        out_type=jax.ShapeDtypeStruct((n, _ACTIONS, _EMB), jnp.float32),
        mesh=mesh,
        scratch_types=[
            pltpu.VMEM((_PLANES, _AG, _PAD), jnp.float32),
            pltpu.VMEM((_NB, _AG), jnp.float32),
            pltpu.VMEM((_NB, _AG, _PAD), jnp.float32),
        ],
        compiler_params=pltpu.CompilerParams(
            use_tc_tiling_on_sc=False, needs_layout_passes=False
        ),
    )
    return run(x, tbl3)


# speedup vs baseline: 1.4104x; 1.1978x over previous
import jax
import jax.numpy as jnp
from jax import lax
from jax.experimental import pallas as pl
from jax.experimental.pallas import tpu as pltpu
from jax.experimental.pallas import tpu_sc as plsc

_DELTA = 10
_ACTIONS = 512
_EMB = 32
_PLANES = 2 * _DELTA + 1

_NC = 2
_NS = 16
_NW = _NC * _NS
_AG = _ACTIONS // _NW
_NB = 64


def _spline_body(x_hbm, tbl_hbm, out_hbm, tbl_v, x_v, out_v, plv, phv, wlv, whv):
    wid = lax.axis_index("s") * _NC + lax.axis_index("c")
    a0 = wid * _AG
    n_total = x_hbm.shape[0]

    pltpu.sync_copy(tbl_hbm.at[:, pl.ds(a0, _AG), :], tbl_v)

    inv_d = jnp.float32(1.0 / _DELTA)

    @pl.loop(0, n_total // _NB)
    def _block(b):
        n0 = b * _NB
        pltpu.sync_copy(x_hbm.at[pl.ds(n0, _NB), pl.ds(a0, _AG)], x_v)

        @plsc.parallel_loop(0, _NB, unroll=2)
        def _prep(i):
            xi = x_v[i, :]
            t = xi * jnp.float32(_DELTA)
            xl_i = t.astype(jnp.int32)
            xh_i = (t + 1.0).astype(jnp.float32).astype(jnp.int32)
            xl_s = xl_i.astype(jnp.float32) / jnp.float32(_DELTA)
            xh_s = xh_i.astype(jnp.float32) / jnp.float32(_DELTA)
            whv[i, :] = (xi - xl_s) / inv_d
            wlv[i, :] = (xh_s - xi) / inv_d
            plv[i, :] = xl_i + _DELTA
            phv[i, :] = jnp.minimum(xh_i + _DELTA, _PLANES - 1)

        @plsc.parallel_loop(0, _NB, unroll=2)
        def _row(i):
            plvec = plv[i, :]
            phvec = phv[i, :]
            wlvec = wlv[i, :]
            whvec = whv[i, :]
            for a in range(_AG):
                pli = plvec[a]
                phi = phvec[a]
                whb = jnp.full((_AG,), whvec[a])
                wlb = jnp.full((_AG,), wlvec[a])
                for h in range(2):
                    bl = tbl_v[pli, a, pl.ds(h * _AG, _AG)]
                    bh = tbl_v[phi, a, pl.ds(h * _AG, _AG)]
                    out_v[i, a, pl.ds(h * _AG, _AG)] = bh * whb + bl * wlb

        pltpu.sync_copy(out_v, out_hbm.at[pl.ds(n0, _NB), pl.ds(a0, _AG), :])


@jax.jit
def kernel(x, table):
    n = x.shape[0]
    tbl3 = table.reshape(_PLANES, _ACTIONS, _EMB)
    mesh = plsc.VectorSubcoreMesh(core_axis_name="c", subcore_axis_name="s")
    run = pl.kernel(
        _spline_body,
        out_type=jax.ShapeDtypeStruct((n, _ACTIONS, _EMB), jnp.float32),
        mesh=mesh,
        scratch_types=[
            pltpu.VMEM((_PLANES, _AG, _EMB), jnp.float32),
            pltpu.VMEM((_NB, _AG), jnp.float32),
            pltpu.VMEM((_NB, _AG, _EMB), jnp.float32),
            pltpu.VMEM((_NB, _AG), jnp.int32),
            pltpu.VMEM((_NB, _AG), jnp.int32),
            pltpu.VMEM((_NB, _AG), jnp.float32),
            pltpu.VMEM((_NB, _AG), jnp.float32),
        ],
        compiler_params=pltpu.CompilerParams(
            use_tc_tiling_on_sc=False, needs_layout_passes=False
        ),
    )
    return run(x, tbl3)


# speedup vs baseline: 1.5281x; 1.0835x over previous
import jax
import jax.numpy as jnp
from jax import lax
from jax.experimental import pallas as pl
from jax.experimental.pallas import tpu as pltpu
from jax.experimental.pallas import tpu_sc as plsc

_DELTA = 10
_ACTIONS = 512
_EMB = 32
_PLANES = 2 * _DELTA + 1

_NC = 2
_NS = 16
_NW = _NC * _NS
_AG = _ACTIONS // _NW
_NB = 64
_NBUF = 2


def _spline_body(
    x_hbm, tbl_hbm, out_hbm, tbl_v, x_v, out_v, plv, phv, wlv, whv, sx, so
):
    wid = lax.axis_index("s") * _NC + lax.axis_index("c")
    a0 = wid * _AG
    n_total = x_hbm.shape[0]
    nblocks = n_total // _NB

    pltpu.sync_copy(tbl_hbm.at[:, pl.ds(a0, _AG), :], tbl_v)

    inv_d = jnp.float32(1.0 / _DELTA)

    def x_copy(b, k):
        return pltpu.make_async_copy(
            x_hbm.at[pl.ds(b * _NB, _NB), pl.ds(a0, _AG)], x_v.at[k], sx.at[k]
        )

    def out_copy(b, k):
        return pltpu.make_async_copy(
            out_v.at[k],
            out_hbm.at[pl.ds(b * _NB, _NB), pl.ds(a0, _AG), :],
            so.at[k],
        )

    for k in range(_NBUF):
        x_copy(k, k).start()

    @pl.loop(0, nblocks, step=_NBUF)
    def _round(r):
        for k in range(_NBUF):
            b = r + k
            x_copy(b, k).wait()

            @pl.when(b >= _NBUF)
            def _():
                out_copy(b - _NBUF, k).wait()

            @plsc.parallel_loop(0, _NB, unroll=2)
            def _prep(i):
                xi = x_v[k, i, :]
                t = xi * jnp.float32(_DELTA)
                xl_i = t.astype(jnp.int32)
                xh_i = (t + 1.0).astype(jnp.float32).astype(jnp.int32)
                xl_s = xl_i.astype(jnp.float32) / jnp.float32(_DELTA)
                xh_s = xh_i.astype(jnp.float32) / jnp.float32(_DELTA)
                whv[i, :] = (xi - xl_s) / inv_d
                wlv[i, :] = (xh_s - xi) / inv_d
                plv[i, :] = xl_i + _DELTA
                phv[i, :] = jnp.minimum(xh_i + _DELTA, _PLANES - 1)

            @plsc.parallel_loop(0, _NB, unroll=2)
            def _row(i):
                plvec = plv[i, :]
                phvec = phv[i, :]
                wlvec = wlv[i, :]
                whvec = whv[i, :]
                for a in range(_AG):
                    pli = plvec[a]
                    phi = phvec[a]
                    whb = jnp.full((_AG,), whvec[a])
                    wlb = jnp.full((_AG,), wlvec[a])
                    for h in range(2):
                        bl = tbl_v[pli, a, pl.ds(h * _AG, _AG)]
                        bh = tbl_v[phi, a, pl.ds(h * _AG, _AG)]
                        out_v[k, i, a, pl.ds(h * _AG, _AG)] = bh * whb + bl * wlb

            out_copy(b, k).start()

            @pl.when(b + _NBUF < nblocks)
            def _():
                x_copy(b + _NBUF, k).start()

    for k in range(_NBUF):
        out_copy(nblocks - _NBUF + k, k).wait()


@jax.jit
def kernel(x, table):
    n = x.shape[0]
    tbl3 = table.reshape(_PLANES, _ACTIONS, _EMB)
    mesh = plsc.VectorSubcoreMesh(core_axis_name="c", subcore_axis_name="s")
    run = pl.kernel(
        _spline_body,
        out_type=jax.ShapeDtypeStruct((n, _ACTIONS, _EMB), jnp.float32),
        mesh=mesh,
        scratch_types=[
            pltpu.VMEM((_PLANES, _AG, _EMB), jnp.float32),
            pltpu.VMEM((_NBUF, _NB, _AG), jnp.float32),
            pltpu.VMEM((_NBUF, _NB, _AG, _EMB), jnp.float32),
            pltpu.VMEM((_NB, _AG), jnp.int32),
            pltpu.VMEM((_NB, _AG), jnp.int32),
            pltpu.VMEM((_NB, _AG), jnp.float32),
            pltpu.VMEM((_NB, _AG), jnp.float32),
            pltpu.SemaphoreType.DMA((_NBUF,)),
            pltpu.SemaphoreType.DMA((_NBUF,)),
        ],
        compiler_params=pltpu.CompilerParams(
            use_tc_tiling_on_sc=False, needs_layout_passes=False
        ),
    )
    return run(x, tbl3)


# speedup vs baseline: 3.0134x; 1.9720x over previous
import jax
import jax.numpy as jnp
from jax import lax
from jax.experimental import pallas as pl
from jax.experimental.pallas import tpu as pltpu
from jax.experimental.pallas import tpu_sc as plsc

_DELTA = 10
_ACTIONS = 512
_EMB = 32
_PLANES = 2 * _DELTA + 1

_NC = 2
_NS = 16
_NW = _NC * _NS
_AG = _ACTIONS // _NW
_NB = 64
_NBUF = 2
_APAD = _AG + 1

_ET = _EMB // 8
_AT = _ACTIONS // 128


def _spline_body(
    x_hbm, tbl_hbm, out_hbm, tbl_v, x_v, out_v, plv, phv, wlv, whv, sx, so
):
    wid = lax.axis_index("s") * _NC + lax.axis_index("c")
    a0 = wid * _AG
    ta = wid // 8
    aoff = (wid % 8) * _AG
    n_total = x_hbm.shape[0]
    nblocks = n_total // _NB

    pltpu.sync_copy(tbl_hbm.at[:, pl.ds(a0, _AG), :], tbl_v)

    iota = lax.iota(jnp.int32, _AG)
    iota_hi8 = lax.shift_right_logical(iota, 3)
    iota_lo8 = lax.bitwise_and(iota, 7)
    inv_d = jnp.float32(1.0 / _DELTA)

    def x_copy(b, k):
        return pltpu.make_async_copy(
            x_hbm.at[pl.ds(b * _NB, _NB), pl.ds(a0, _AG)], x_v.at[k], sx.at[k]
        )

    def out_copy(b, k):
        return pltpu.make_async_copy(
            out_v.at[k, :, :, :, pl.ds(0, _AG)],
            out_hbm.at[pl.ds(b * _NB, _NB), :, ta, :, pl.ds(aoff, _AG)],
            so.at[k],
        )

    for k in range(_NBUF):
        x_copy(k, k).start()

    @pl.loop(0, nblocks, step=_NBUF)
    def _round(r):
        for k in range(_NBUF):
            b = r + k
            x_copy(b, k).wait()

            @pl.when(b >= _NBUF)
            def _():
                out_copy(b - _NBUF, k).wait()

            @plsc.parallel_loop(0, _NB, unroll=2)
            def _prep(i):
                xi = x_v[k, i, :]
                t = xi * jnp.float32(_DELTA)
                xl_i = t.astype(jnp.int32)
                xh_i = (t + 1.0).astype(jnp.float32).astype(jnp.int32)
                xl_s = xl_i.astype(jnp.float32) / jnp.float32(_DELTA)
                xh_s = xh_i.astype(jnp.float32) / jnp.float32(_DELTA)
                whv[i, :] = (xi - xl_s) / inv_d
                wlv[i, :] = (xh_s - xi) / inv_d
                plv[i, :] = xl_i + _DELTA
                phv[i, :] = jnp.minimum(xh_i + _DELTA, _PLANES - 1)

            @plsc.parallel_loop(0, _NB, unroll=2)
            def _row(i):
                plvec = plv[i, :]
                phvec = phv[i, :]
                wlvec = wlv[i, :]
                whvec = whv[i, :]
                for a in range(_AG):
                    pli = plvec[a]
                    phi = phvec[a]
                    whb = jnp.full((_AG,), whvec[a])
                    wlb = jnp.full((_AG,), wlvec[a])
                    for h in range(2):
                        bl = tbl_v[pli, a, pl.ds(h * _AG, _AG)]
                        bh = tbl_v[phi, a, pl.ds(h * _AG, _AG)]
                        val = bh * whb + bl * wlb
                        plsc.store_scatter(
                            out_v,
                            [
                                jnp.full((_AG,), k, jnp.int32),
                                jnp.full((_AG,), i, jnp.int32),
                                iota_hi8 + (h * 2),
                                iota_lo8,
                                jnp.full((_AG,), a, jnp.int32),
                            ],
                            val,
                        )

            out_copy(b, k).start()

            @pl.when(b + _NBUF < nblocks)
            def _():
                x_copy(b + _NBUF, k).start()

    for k in range(_NBUF):
        out_copy(nblocks - _NBUF + k, k).wait()


@jax.jit
def kernel(x, table):
    n = x.shape[0]
    tbl3 = table.reshape(_PLANES, _ACTIONS, _EMB)
    mesh = plsc.VectorSubcoreMesh(core_axis_name="c", subcore_axis_name="s")
    run = pl.kernel(
        _spline_body,
        out_type=jax.ShapeDtypeStruct((n, _ET, _AT, 8, 128), jnp.float32),
        mesh=mesh,
        scratch_types=[
            pltpu.VMEM((_PLANES, _AG, _EMB), jnp.float32),
            pltpu.VMEM((_NBUF, _NB, _AG), jnp.float32),
            pltpu.VMEM((_NBUF, _NB, _ET, 8, _APAD), jnp.float32),
            pltpu.VMEM((_NB, _AG), jnp.int32),
            pltpu.VMEM((_NB, _AG), jnp.int32),
            pltpu.VMEM((_NB, _AG), jnp.float32),
            pltpu.VMEM((_NB, _AG), jnp.float32),
            pltpu.SemaphoreType.DMA((_NBUF,)),
            pltpu.SemaphoreType.DMA((_NBUF,)),
        ],
        compiler_params=pltpu.CompilerParams(
            use_tc_tiling_on_sc=False, needs_layout_passes=False
        ),
    )
    h5 = run(x, tbl3)
    ht = h5.transpose(0, 1, 3, 2, 4)
    he = ht.reshape(n, _EMB, _ACTIONS)
    return he.swapaxes(1, 2)
